# Initial kernel scaffold; baseline (speedup 1.0000x reference)
import jax
import jax.numpy as jnp
from jax.experimental import pallas as pl


def kernel(x, edge_index, edge_type, W1, root1, b1, W2, root2, b2):
    raise NotImplementedError("write your pallas kernel here")


# speedup vs baseline: 12.7319x; 12.7319x over previous
import functools

import jax
import jax.numpy as jnp
from jax import lax
from jax.experimental import pallas as pl
from jax.experimental.pallas import tpu as pltpu
from jax.experimental.pallas import tpu_sc as plsc

N = 10000
E = 320000
D_IN = 128
D_HID = 64
D_OUT = 128
R = 8
RN = R * N

NC = 2
NS = 16
NW = NC * NS
L = 16

EPT = E // NW
G = 80
STEPS = EPT // G
ROWS_PT = N // NS
ZROWS = 125

_mesh = plsc.VectorSubcoreMesh(core_axis_name="c", subcore_axis_name="s")
_sc_params = pltpu.CompilerParams(use_tc_tiling_on_sc=False,
                                  needs_layout_passes=False)


DEG_SLICE = RN // NS


def _deg_body(dst_hbm, et_hbm, deg_out, dstb, etb, idxb, onesb, zb, acc):
  c = lax.axis_index("c")
  s = lax.axis_index("s")
  wid = s * NC + c
  base = wid * EPT

  z16 = jnp.zeros((L,), jnp.float32)

  def zfill(i, _):
    zb[pl.ds(i * L, L)] = z16
    return 0

  lax.fori_loop(0, DEG_SLICE // L, zfill, 0)
  ones16 = jnp.ones((L,), jnp.float32)
  for k in range(G // L):
    onesb[pl.ds(k * L, L)] = ones16

  pltpu.sync_copy(zb, acc.at[pl.ds(s * DEG_SLICE, DEG_SLICE)])
  plsc.subcore_barrier()

  def step(j, _):
    off = base + j * G
    pltpu.sync_copy(dst_hbm.at[pl.ds(off, G)], dstb)
    pltpu.sync_copy(et_hbm.at[pl.ds(off, G)], etb)
    for k in range(G // L):
      d16 = dstb[pl.ds(k * L, L)]
      t16 = etb[pl.ds(k * L, L)]
      idxb[0, pl.ds(k * L, L)] = t16 * N + d16
    pltpu.sync_copy(onesb, acc.at[idxb.at[0]], add=True)
    return 0

  lax.fori_loop(0, STEPS, step, 0)
  plsc.subcore_barrier()
  pltpu.sync_copy(acc.at[pl.ds(s * DEG_SLICE, DEG_SLICE)], zb)
  pltpu.sync_copy(zb, deg_out.at[c, pl.ds(s * DEG_SLICE, DEG_SLICE)])


_deg_call = functools.partial(
    pl.kernel,
    out_type=jax.ShapeDtypeStruct((NC, RN), jnp.float32),
    mesh=_mesh,
    scratch_types=[
        pltpu.VMEM((G,), jnp.int32),
        pltpu.VMEM((G,), jnp.int32),
        pltpu.VMEM((1, G), jnp.int32),
        pltpu.VMEM((G,), jnp.float32),
        pltpu.VMEM((DEG_SLICE,), jnp.float32),
        pltpu.VMEM_SHARED((RN,), jnp.float32),
    ],
    compiler_params=_sc_params,
)(_deg_body)


def _edge_body(d_feat, with_table, hall_hbm, src_hbm, dst_hbm, et_hbm,
               recip_hbm, acc_out, norm_out, srcb, dstb, etb, gidx, sidx,
               nidx, normb, rows, zb, acc):
  c = lax.axis_index("c")
  s = lax.axis_index("s")
  wid = s * NC + c
  base = wid * EPT
  nchunk = d_feat // L

  z16 = jnp.zeros((L,), jnp.float32)

  def zfill(i, _):
    for c4 in range(nchunk):
      zb[i, pl.ds(c4 * L, L)] = z16
    return 0

  lax.fori_loop(0, ZROWS, zfill, 0)

  for i in range(ROWS_PT // ZROWS):
    pltpu.sync_copy(zb, acc.at[pl.ds(s * ROWS_PT + i * ZROWS, ZROWS), :])
  plsc.subcore_barrier()

  def step(j, _):
    off = base + j * G
    pltpu.sync_copy(src_hbm.at[pl.ds(off, G)], srcb)
    pltpu.sync_copy(dst_hbm.at[pl.ds(off, G)], dstb)
    pltpu.sync_copy(et_hbm.at[pl.ds(off, G)], etb)
    for k in range(G // L):
      s16 = srcb[pl.ds(k * L, L)]
      d16 = dstb[pl.ds(k * L, L)]
      t16 = etb[pl.ds(k * L, L)]
      gidx[0, pl.ds(k * L, L)] = t16 * N + s16
      sidx[0, pl.ds(k * L, L)] = d16
      if with_table:
        nidx[0, pl.ds(k * L, L)] = t16 * N + d16
    if with_table:
      pltpu.sync_copy(recip_hbm.at[nidx.at[0]], normb)
      pltpu.sync_copy(normb, norm_out.at[pl.ds(off, G)])
    else:
      pltpu.sync_copy(recip_hbm.at[pl.ds(off, G)], normb)
    pltpu.sync_copy(hall_hbm.at[gidx.at[0]], rows)

    def rowscale(r, _):
      sp = plsc.load_gather(normb, [jnp.full((L,), r, jnp.int32)])
      for c4 in range(nchunk):
        rows[r, pl.ds(c4 * L, L)] = rows[r, pl.ds(c4 * L, L)] * sp
      return 0

    lax.fori_loop(0, G, rowscale, 0)
    pltpu.sync_copy(rows, acc.at[sidx.at[0]], add=True)
    return 0

  lax.fori_loop(0, STEPS, step, 0)
  plsc.subcore_barrier()
  for i in range(ROWS_PT // ZROWS):
    pltpu.sync_copy(acc.at[pl.ds(s * ROWS_PT + i * ZROWS, ZROWS), :], zb)
    pltpu.sync_copy(zb, acc_out.at[c, pl.ds(s * ROWS_PT + i * ZROWS, ZROWS), :])


def _edge_call(d_feat, with_table):
  scratch = [
      pltpu.VMEM((G,), jnp.int32),
      pltpu.VMEM((G,), jnp.int32),
      pltpu.VMEM((G,), jnp.int32),
      pltpu.VMEM((1, G), jnp.int32),
      pltpu.VMEM((1, G), jnp.int32),
      pltpu.VMEM((1, G), jnp.int32),
      pltpu.VMEM((G,), jnp.float32),
      pltpu.VMEM((G, d_feat), jnp.float32),
      pltpu.VMEM((ZROWS, d_feat), jnp.float32),
      pltpu.VMEM_SHARED((N, d_feat), jnp.float32),
  ]
  out_type = [jax.ShapeDtypeStruct((NC, N, d_feat), jnp.float32)]
  if with_table:
    out_type.append(jax.ShapeDtypeStruct((E,), jnp.float32))

  def body(*args):
    if with_table:
      (hall, src, dst, et, recip, acc_out, norm_out,
       srcb, dstb, etb, gidx, sidx, nidx, normb, rows, zb, acc) = args
    else:
      (hall, src, dst, et, recip, acc_out,
       srcb, dstb, etb, gidx, sidx, nidx, normb, rows, zb, acc) = args
      norm_out = None
    _edge_body(d_feat, with_table, hall, src, dst, et, recip, acc_out,
               norm_out, srcb, dstb, etb, gidx, sidx, nidx, normb, rows, zb,
               acc)

  return pl.kernel(body, out_type=out_type, mesh=_mesh,
                   scratch_types=scratch, compiler_params=_sc_params)


_edge1_call = _edge_call(D_HID, True)
_edge2_call = _edge_call(D_OUT, False)


NB = 10
BN = N // NB


def _tc1_body(deg_ref, x_ref, w_ref, root_ref, b_ref,
              hall_ref, xr_ref, recip_ref):
  xb = x_ref[...]
  for r in range(R):
    hall_ref[r] = jnp.dot(xb, w_ref[r], preferred_element_type=jnp.float32)
  xr_ref[...] = (jnp.dot(xb, root_ref[...], preferred_element_type=jnp.float32)
                 + b_ref[...])

  @pl.when(pl.program_id(0) == 0)
  def _():
    d = jnp.sum(deg_ref[...], axis=0)
    recip_ref[...] = 1.0 / jnp.maximum(d, 1.0)


def _tc1(deg32, x, W1, root1, b1):
  return pl.pallas_call(
      _tc1_body,
      grid=(NB,),
      in_specs=[
          pl.BlockSpec((NC, RN), lambda i: (0, 0)),
          pl.BlockSpec((BN, D_IN), lambda i: (i, 0)),
          pl.BlockSpec((R, D_IN, D_HID), lambda i: (0, 0, 0)),
          pl.BlockSpec((D_IN, D_HID), lambda i: (0, 0)),
          pl.BlockSpec((1, D_HID), lambda i: (0, 0)),
      ],
      out_specs=[
          pl.BlockSpec((R, BN, D_HID), lambda i: (0, i, 0)),
          pl.BlockSpec((BN, D_HID), lambda i: (i, 0)),
          pl.BlockSpec((RN,), lambda i: (0,)),
      ],
      out_shape=[
          jax.ShapeDtypeStruct((R, N, D_HID), jnp.float32),
          jax.ShapeDtypeStruct((N, D_HID), jnp.float32),
          jax.ShapeDtypeStruct((RN,), jnp.float32),
      ],
      compiler_params=pltpu.CompilerParams(vmem_limit_bytes=100 * 1024 * 1024),
  )(deg32, x, W1, root1, b1)


def _tc2_body(acc_ref, xr_ref, w_ref, root_ref, b_ref, hall_ref, xr2_ref):
  hb = jnp.maximum(acc_ref[0] + acc_ref[1] + xr_ref[...], 0.0)
  for r in range(R):
    hall_ref[r] = jnp.dot(hb, w_ref[r], preferred_element_type=jnp.float32)
  xr2_ref[...] = (jnp.dot(hb, root_ref[...], preferred_element_type=jnp.float32)
                  + b_ref[...])


def _tc2(acc1, xr1, W2, root2, b2):
  return pl.pallas_call(
      _tc2_body,
      grid=(NB,),
      in_specs=[
          pl.BlockSpec((2, BN, D_HID), lambda i: (0, i, 0)),
          pl.BlockSpec((BN, D_HID), lambda i: (i, 0)),
          pl.BlockSpec((R, D_HID, D_OUT), lambda i: (0, 0, 0)),
          pl.BlockSpec((D_HID, D_OUT), lambda i: (0, 0)),
          pl.BlockSpec((1, D_OUT), lambda i: (0, 0)),
      ],
      out_specs=[
          pl.BlockSpec((R, BN, D_OUT), lambda i: (0, i, 0)),
          pl.BlockSpec((BN, D_OUT), lambda i: (i, 0)),
      ],
      out_shape=[
          jax.ShapeDtypeStruct((R, N, D_OUT), jnp.float32),
          jax.ShapeDtypeStruct((N, D_OUT), jnp.float32),
      ],
  )(acc1, xr1, W2, root2, b2)


def _tc3_body(acc_ref, xr_ref, out_ref):
  out_ref[...] = acc_ref[0] + acc_ref[1] + xr_ref[...]


def _tc3(acc2, xr2):
  return pl.pallas_call(
      _tc3_body,
      grid=(NB,),
      in_specs=[
          pl.BlockSpec((2, BN, D_OUT), lambda i: (0, i, 0)),
          pl.BlockSpec((BN, D_OUT), lambda i: (i, 0)),
      ],
      out_specs=pl.BlockSpec((BN, D_OUT), lambda i: (i, 0)),
      out_shape=jax.ShapeDtypeStruct((N, D_OUT), jnp.float32),
  )(acc2, xr2)


@jax.jit
def kernel(x, edge_index, edge_type, W1, root1, b1, W2, root2, b2):
  src = edge_index[0]
  dst = edge_index[1]

  deg32 = _deg_call(dst, edge_type)
  hall1, xr1, recip = _tc1(deg32, x, W1, root1, b1.reshape(1, D_HID))
  acc1, norm_e = _edge1_call(hall1.reshape(RN, D_HID), src, dst, edge_type,
                             recip)
  hall2, xr2 = _tc2(acc1, xr1, W2, root2, b2.reshape(1, D_OUT))
  (acc2,) = _edge2_call(hall2.reshape(RN, D_OUT), src, dst, edge_type, norm_e)
  return _tc3(acc2, xr2)


# speedup vs baseline: 25.9333x; 2.0369x over previous
import functools

import jax
import jax.numpy as jnp
from jax import lax
from jax.experimental import pallas as pl
from jax.experimental.pallas import tpu as pltpu
from jax.experimental.pallas import tpu_sc as plsc

N = 10000
E = 320000
D_IN = 128
D_HID = 64
D_OUT = 128
R = 8
RN = R * N

NC = 2
NS = 16
NW = NC * NS
L = 16

EPT = E // NW
G = 80
STEPS = EPT // G
ROWS_PT = N // NS
ZROWS = 125

_mesh = plsc.VectorSubcoreMesh(core_axis_name="c", subcore_axis_name="s")
_sc_params = pltpu.CompilerParams(use_tc_tiling_on_sc=False,
                                  needs_layout_passes=False)


DEG_SLICE = RN // NS


def _deg_body(dst_hbm, et_hbm, deg_out, dstb, etb, idxb, onesb, zb, acc):
  c = lax.axis_index("c")
  s = lax.axis_index("s")
  wid = s * NC + c
  base = wid * EPT

  z16 = jnp.zeros((L,), jnp.float32)

  def zfill(i, _):
    zb[pl.ds(i * L, L)] = z16
    return 0

  lax.fori_loop(0, DEG_SLICE // L, zfill, 0)
  ones16 = jnp.ones((L,), jnp.float32)
  for k in range(G // L):
    onesb[pl.ds(k * L, L)] = ones16

  pltpu.sync_copy(zb, acc.at[pl.ds(s * DEG_SLICE, DEG_SLICE)])
  plsc.subcore_barrier()

  def step(j, _):
    off = base + j * G
    pltpu.sync_copy(dst_hbm.at[pl.ds(off, G)], dstb)
    pltpu.sync_copy(et_hbm.at[pl.ds(off, G)], etb)
    for k in range(G // L):
      d16 = dstb[pl.ds(k * L, L)]
      t16 = etb[pl.ds(k * L, L)]
      idxb[0, pl.ds(k * L, L)] = t16 * N + d16
    pltpu.sync_copy(onesb, acc.at[idxb.at[0]], add=True)
    return 0

  lax.fori_loop(0, STEPS, step, 0)
  plsc.subcore_barrier()
  pltpu.sync_copy(acc.at[pl.ds(s * DEG_SLICE, DEG_SLICE)], zb)
  pltpu.sync_copy(zb, deg_out.at[c, pl.ds(s * DEG_SLICE, DEG_SLICE)])


_deg_call = functools.partial(
    pl.kernel,
    out_type=jax.ShapeDtypeStruct((NC, RN), jnp.float32),
    mesh=_mesh,
    scratch_types=[
        pltpu.VMEM((G,), jnp.int32),
        pltpu.VMEM((G,), jnp.int32),
        pltpu.VMEM((1, G), jnp.int32),
        pltpu.VMEM((G,), jnp.float32),
        pltpu.VMEM((DEG_SLICE,), jnp.float32),
        pltpu.VMEM_SHARED((RN,), jnp.float32),
    ],
    compiler_params=_sc_params,
)(_deg_body)


CH = 2000
GPC = CH // G
NCHK = EPT // CH
PAIRS = (GPC - 1) // 2


def _edge_body(d_feat, with_table, hall_hbm, src_hbm, dst_hbm, et_hbm,
               recip_hbm, acc_out, norm_out, srcc, dstc, etc_, gidx, sidx,
               nidx, normc, rows0, rows1, zb, acc,
               sg0, sg1, ss0, ss1, sn):
  c = lax.axis_index("c")
  s = lax.axis_index("s")
  wid = s * NC + c
  base = wid * EPT
  nchunk = d_feat // L

  z16 = jnp.zeros((L,), jnp.float32)

  def zfill(i, _):
    for c4 in range(nchunk):
      zb[i, pl.ds(c4 * L, L)] = z16
    return 0

  lax.fori_loop(0, ZROWS, zfill, 0)

  for i in range(ROWS_PT // ZROWS):
    pltpu.sync_copy(zb, acc.at[pl.ds(s * ROWS_PT + i * ZROWS, ZROWS), :])
  plsc.subcore_barrier()

  def fire_g(g, rows, sem):
    pltpu.async_copy(hall_hbm.at[gidx.at[pl.ds(g * G, G)]], rows, sem)

  def wait_g(rows, sem):
    pltpu.make_async_copy(hall_hbm.at[gidx.at[pl.ds(0, G)]], rows, sem).wait()

  def fire_s(g, rows, sem):
    pltpu.async_copy(rows, acc.at[sidx.at[g]], sem, add=True)

  def wait_s(rows, sem):
    pltpu.make_async_copy(rows, acc.at[sidx.at[0]], sem).wait()

  def scale(rows, goff):
    def rowscale(r, _):
      for u in range(2):
        sp = plsc.load_gather(normc, [jnp.full((L,), goff + 2 * r + u,
                                               jnp.int32)])
        for c4 in range(nchunk):
          rows[2 * r + u, pl.ds(c4 * L, L)] = (
              rows[2 * r + u, pl.ds(c4 * L, L)] * sp)
      return 0

    lax.fori_loop(0, G // 2, rowscale, 0)

  def chunk(ci, _):
    coff = base + ci * CH
    pltpu.sync_copy(src_hbm.at[pl.ds(coff, CH)], srcc)
    pltpu.sync_copy(dst_hbm.at[pl.ds(coff, CH)], dstc)
    pltpu.sync_copy(et_hbm.at[pl.ds(coff, CH)], etc_)

    def cidx(g, _):
      for q in range(G // L):
        o = pl.ds(g * G + q * L, L)
        s16 = srcc[o]
        d16 = dstc[o]
        t16 = etc_[o]
        gidx[pl.ds(g * G + q * L, L)] = t16 * N + s16
        sidx[g, pl.ds(q * L, L)] = d16
        if with_table:
          nidx[pl.ds(g * G + q * L, L)] = t16 * N + d16
      return 0

    lax.fori_loop(0, GPC, cidx, 0)

    if with_table:
      for g in range(GPC):
        pltpu.async_copy(recip_hbm.at[nidx.at[pl.ds(g * G, G)]],
                         normc.at[pl.ds(g * G, G)], sn)
    else:
      pltpu.sync_copy(recip_hbm.at[pl.ds(coff, CH)], normc)

    fire_g(0, rows0, sg0)
    fire_g(1, rows1, sg1)

    if with_table:
      for g in range(GPC):
        pltpu.make_async_copy(recip_hbm.at[nidx.at[pl.ds(0, G)]],
                              normc.at[pl.ds(0, G)], sn).wait()
      pltpu.sync_copy(normc, norm_out.at[pl.ds(coff, CH)])

    def pair(i, _):
      g0 = 2 * i
      g1 = 2 * i + 1
      wait_g(rows0, sg0)
      scale(rows0, g0 * G)
      fire_s(g0, rows0, ss0)
      wait_g(rows1, sg1)
      scale(rows1, g1 * G)
      fire_s(g1, rows1, ss1)

      @pl.when(i < PAIRS - 1)
      def _():
        wait_s(rows0, ss0)
        fire_g(g0 + 2, rows0, sg0)
        wait_s(rows1, ss1)
        fire_g(g1 + 2, rows1, sg1)

      return 0

    lax.fori_loop(0, PAIRS, pair, 0)

    wait_s(rows0, ss0)
    fire_g(GPC - 1, rows0, sg0)
    wait_g(rows0, sg0)
    scale(rows0, (GPC - 1) * G)
    fire_s(GPC - 1, rows0, ss0)
    wait_s(rows0, ss0)
    wait_s(rows1, ss1)
    return 0

  lax.fori_loop(0, NCHK, chunk, 0)
  plsc.subcore_barrier()
  for i in range(ROWS_PT // ZROWS):
    pltpu.sync_copy(acc.at[pl.ds(s * ROWS_PT + i * ZROWS, ZROWS), :], zb)
    pltpu.sync_copy(zb, acc_out.at[c, pl.ds(s * ROWS_PT + i * ZROWS, ZROWS), :])


def _edge_call(d_feat, with_table):
  scratch = [
      pltpu.VMEM((CH,), jnp.int32),
      pltpu.VMEM((CH,), jnp.int32),
      pltpu.VMEM((CH,), jnp.int32),
      pltpu.VMEM((CH,), jnp.int32),
      pltpu.VMEM((GPC, G), jnp.int32),
      pltpu.VMEM((CH,), jnp.int32),
      pltpu.VMEM((CH,), jnp.float32),
      pltpu.VMEM((G, d_feat), jnp.float32),
      pltpu.VMEM((G, d_feat), jnp.float32),
      pltpu.VMEM((ZROWS, d_feat), jnp.float32),
      pltpu.VMEM_SHARED((N, d_feat), jnp.float32),
      pltpu.SemaphoreType.DMA,
      pltpu.SemaphoreType.DMA,
      pltpu.SemaphoreType.DMA,
      pltpu.SemaphoreType.DMA,
      pltpu.SemaphoreType.DMA,
  ]
  out_type = [jax.ShapeDtypeStruct((NC, N, d_feat), jnp.float32)]
  if with_table:
    out_type.append(jax.ShapeDtypeStruct((E,), jnp.float32))

  def body(*args):
    if with_table:
      (hall, src, dst, et, recip, acc_out, norm_out, *rest) = args
    else:
      (hall, src, dst, et, recip, acc_out, *rest) = args
      norm_out = None
    _edge_body(d_feat, with_table, hall, src, dst, et, recip, acc_out,
               norm_out, *rest)

  return pl.kernel(body, out_type=out_type, mesh=_mesh,
                   scratch_types=scratch, compiler_params=_sc_params)


_edge1_call = _edge_call(D_HID, True)
_edge2_call = _edge_call(D_OUT, False)


NB = 10
BN = N // NB


def _tc1_body(deg_ref, x_ref, w_ref, root_ref, b_ref,
              hall_ref, xr_ref, recip_ref):
  xb = x_ref[...]
  for r in range(R):
    hall_ref[r] = jnp.dot(xb, w_ref[r], preferred_element_type=jnp.float32)
  xr_ref[...] = (jnp.dot(xb, root_ref[...], preferred_element_type=jnp.float32)
                 + b_ref[...])

  @pl.when(pl.program_id(0) == 0)
  def _():
    d = jnp.sum(deg_ref[...], axis=0)
    recip_ref[...] = 1.0 / jnp.maximum(d, 1.0)


def _tc1(deg32, x, W1, root1, b1):
  return pl.pallas_call(
      _tc1_body,
      grid=(NB,),
      in_specs=[
          pl.BlockSpec((NC, RN), lambda i: (0, 0)),
          pl.BlockSpec((BN, D_IN), lambda i: (i, 0)),
          pl.BlockSpec((R, D_IN, D_HID), lambda i: (0, 0, 0)),
          pl.BlockSpec((D_IN, D_HID), lambda i: (0, 0)),
          pl.BlockSpec((1, D_HID), lambda i: (0, 0)),
      ],
      out_specs=[
          pl.BlockSpec((R, BN, D_HID), lambda i: (0, i, 0)),
          pl.BlockSpec((BN, D_HID), lambda i: (i, 0)),
          pl.BlockSpec((RN,), lambda i: (0,)),
      ],
      out_shape=[
          jax.ShapeDtypeStruct((R, N, D_HID), jnp.float32),
          jax.ShapeDtypeStruct((N, D_HID), jnp.float32),
          jax.ShapeDtypeStruct((RN,), jnp.float32),
      ],
      compiler_params=pltpu.CompilerParams(vmem_limit_bytes=100 * 1024 * 1024),
  )(deg32, x, W1, root1, b1)


def _tc2_body(acc_ref, xr_ref, w_ref, root_ref, b_ref, hall_ref, xr2_ref):
  hb = jnp.maximum(acc_ref[0] + acc_ref[1] + xr_ref[...], 0.0)
  for r in range(R):
    hall_ref[r] = jnp.dot(hb, w_ref[r], preferred_element_type=jnp.float32)
  xr2_ref[...] = (jnp.dot(hb, root_ref[...], preferred_element_type=jnp.float32)
                  + b_ref[...])


def _tc2(acc1, xr1, W2, root2, b2):
  return pl.pallas_call(
      _tc2_body,
      grid=(NB,),
      in_specs=[
          pl.BlockSpec((2, BN, D_HID), lambda i: (0, i, 0)),
          pl.BlockSpec((BN, D_HID), lambda i: (i, 0)),
          pl.BlockSpec((R, D_HID, D_OUT), lambda i: (0, 0, 0)),
          pl.BlockSpec((D_HID, D_OUT), lambda i: (0, 0)),
          pl.BlockSpec((1, D_OUT), lambda i: (0, 0)),
      ],
      out_specs=[
          pl.BlockSpec((R, BN, D_OUT), lambda i: (0, i, 0)),
          pl.BlockSpec((BN, D_OUT), lambda i: (i, 0)),
      ],
      out_shape=[
          jax.ShapeDtypeStruct((R, N, D_OUT), jnp.float32),
          jax.ShapeDtypeStruct((N, D_OUT), jnp.float32),
      ],
  )(acc1, xr1, W2, root2, b2)


def _tc3_body(acc_ref, xr_ref, out_ref):
  out_ref[...] = acc_ref[0] + acc_ref[1] + xr_ref[...]


def _tc3(acc2, xr2):
  return pl.pallas_call(
      _tc3_body,
      grid=(NB,),
      in_specs=[
          pl.BlockSpec((2, BN, D_OUT), lambda i: (0, i, 0)),
          pl.BlockSpec((BN, D_OUT), lambda i: (i, 0)),
      ],
      out_specs=pl.BlockSpec((BN, D_OUT), lambda i: (i, 0)),
      out_shape=jax.ShapeDtypeStruct((N, D_OUT), jnp.float32),
  )(acc2, xr2)


@jax.jit
def kernel(x, edge_index, edge_type, W1, root1, b1, W2, root2, b2):
  src = edge_index[0]
  dst = edge_index[1]

  deg32 = _deg_call(dst, edge_type)
  hall1, xr1, recip = _tc1(deg32, x, W1, root1, b1.reshape(1, D_HID))
  acc1, norm_e = _edge1_call(hall1.reshape(RN, D_HID), src, dst, edge_type,
                             recip)
  hall2, xr2 = _tc2(acc1, xr1, W2, root2, b2.reshape(1, D_OUT))
  (acc2,) = _edge2_call(hall2.reshape(RN, D_OUT), src, dst, edge_type, norm_e)
  return _tc3(acc2, xr2)


# speedup vs baseline: 31.6772x; 1.2215x over previous
import functools

import jax
import jax.numpy as jnp
from jax import lax
from jax.experimental import pallas as pl
from jax.experimental.pallas import tpu as pltpu
from jax.experimental.pallas import tpu_sc as plsc

N = 10000
E = 320000
D_IN = 128
D_HID = 64
D_OUT = 128
R = 8
RN = R * N

NC = 2
NS = 16
NW = NC * NS
L = 16

EPT = E // NW
G = 80
STEPS = EPT // G
ROWS_PT = N // NS
ZROWS = 125

_mesh = plsc.VectorSubcoreMesh(core_axis_name="c", subcore_axis_name="s")
_sc_params = pltpu.CompilerParams(use_tc_tiling_on_sc=False,
                                  needs_layout_passes=False)


CH = 2000
GPC = CH // G
NCHK = EPT // CH
PAIRS = (GPC - 1) // 2
DEG_SLICE = 5008
RNP = NS * DEG_SLICE
EPS = E // NS


def _edge_body(d_feat, with_table, hall_hbm, src_hbm, dst_hbm, et_hbm,
               recip_hbm, acc_out, norm_out, srcc, dstc, etc_, gidx, sidx,
               nidx, normc, rows0, rows1, zb, acc, dbuf, onesb, degacc,
               sg0, sg1, ss0, ss1, sn):
  c = lax.axis_index("c")
  s = lax.axis_index("s")
  wid = s * NC + c
  base = wid * EPT
  nchunk = d_feat // L

  z16 = jnp.zeros((L,), jnp.float32)

  def zfill(i, _):
    for c4 in range(nchunk):
      zb[i, pl.ds(c4 * L, L)] = z16
    return 0

  lax.fori_loop(0, ZROWS, zfill, 0)

  for i in range(ROWS_PT // ZROWS):
    pltpu.sync_copy(zb, acc.at[pl.ds(s * ROWS_PT + i * ZROWS, ZROWS), :])

  if with_table:
    def dzfill(i, _):
      dbuf[pl.ds(i * L, L)] = z16
      return 0

    lax.fori_loop(0, DEG_SLICE // L, dzfill, 0)
    ones16 = jnp.ones((L,), jnp.float32)
    for k in range(G // L):
      onesb[pl.ds(k * L, L)] = ones16
    dslice = pl.ds(s * DEG_SLICE, DEG_SLICE)
    pltpu.sync_copy(dbuf, degacc.at[dslice])
    plsc.subcore_barrier()

    dbase = s * EPS

    def degchunk(ci, _):
      coff = dbase + ci * CH
      pltpu.sync_copy(dst_hbm.at[pl.ds(coff, CH)], dstc)
      pltpu.sync_copy(et_hbm.at[pl.ds(coff, CH)], etc_)

      def didxf(g, _):
        for q in range(G // L):
          o = pl.ds(g * G + q * L, L)
          sidx[g, pl.ds(q * L, L)] = etc_[o] * N + dstc[o]
        return 0

      lax.fori_loop(0, GPC, didxf, 0)
      for g in range(GPC):
        pltpu.async_copy(onesb, degacc.at[sidx.at[g]], sn, add=True)
      for g in range(GPC):
        pltpu.make_async_copy(onesb, degacc.at[sidx.at[0]], sn).wait()
      return 0

    lax.fori_loop(0, EPS // CH, degchunk, 0)
    plsc.subcore_barrier()

    pltpu.sync_copy(degacc.at[dslice], dbuf)

    def recipf(i, _):
      v = dbuf[pl.ds(i * L, L)]
      dbuf[pl.ds(i * L, L)] = 1.0 / jnp.maximum(v, 1.0)
      return 0

    lax.fori_loop(0, DEG_SLICE // L, recipf, 0)
    pltpu.sync_copy(dbuf, degacc.at[dslice])
  plsc.subcore_barrier()

  def fire_g(g, rows, sem):
    pltpu.async_copy(hall_hbm.at[gidx.at[pl.ds(g * G, G)]], rows, sem)

  def wait_g(rows, sem):
    pltpu.make_async_copy(hall_hbm.at[gidx.at[pl.ds(0, G)]], rows, sem).wait()

  def fire_s(g, rows, sem):
    pltpu.async_copy(rows, acc.at[sidx.at[g]], sem, add=True)

  def wait_s(rows, sem):
    pltpu.make_async_copy(rows, acc.at[sidx.at[0]], sem).wait()

  def scale(rows, goff):
    def rowscale(r, _):
      for u in range(2):
        sp = plsc.load_gather(normc, [jnp.full((L,), goff + 2 * r + u,
                                               jnp.int32)])
        for c4 in range(nchunk):
          rows[2 * r + u, pl.ds(c4 * L, L)] = (
              rows[2 * r + u, pl.ds(c4 * L, L)] * sp)
      return 0

    lax.fori_loop(0, G // 2, rowscale, 0)

  def chunk(ci, _):
    coff = base + ci * CH
    pltpu.sync_copy(src_hbm.at[pl.ds(coff, CH)], srcc)
    pltpu.sync_copy(dst_hbm.at[pl.ds(coff, CH)], dstc)
    pltpu.sync_copy(et_hbm.at[pl.ds(coff, CH)], etc_)

    def cidx(g, _):
      for q in range(G // L):
        o = pl.ds(g * G + q * L, L)
        s16 = srcc[o]
        d16 = dstc[o]
        t16 = etc_[o]
        gidx[pl.ds(g * G + q * L, L)] = t16 * N + s16
        sidx[g, pl.ds(q * L, L)] = d16
        if with_table:
          nidx[pl.ds(g * G + q * L, L)] = t16 * N + d16
      return 0

    lax.fori_loop(0, GPC, cidx, 0)

    if with_table:
      for g in range(GPC):
        pltpu.async_copy(degacc.at[nidx.at[pl.ds(g * G, G)]],
                         normc.at[pl.ds(g * G, G)], sn)
    else:
      pltpu.sync_copy(recip_hbm.at[pl.ds(coff, CH)], normc)

    fire_g(0, rows0, sg0)
    fire_g(1, rows1, sg1)

    if with_table:
      for g in range(GPC):
        pltpu.make_async_copy(degacc.at[nidx.at[pl.ds(0, G)]],
                              normc.at[pl.ds(0, G)], sn).wait()
      pltpu.sync_copy(normc, norm_out.at[pl.ds(coff, CH)])

    def pair(i, _):
      g0 = 2 * i
      g1 = 2 * i + 1
      wait_g(rows0, sg0)
      scale(rows0, g0 * G)
      fire_s(g0, rows0, ss0)
      wait_g(rows1, sg1)
      scale(rows1, g1 * G)
      fire_s(g1, rows1, ss1)

      @pl.when(i < PAIRS - 1)
      def _():
        wait_s(rows0, ss0)
        fire_g(g0 + 2, rows0, sg0)
        wait_s(rows1, ss1)
        fire_g(g1 + 2, rows1, sg1)

      return 0

    lax.fori_loop(0, PAIRS, pair, 0)

    wait_s(rows0, ss0)
    fire_g(GPC - 1, rows0, sg0)
    wait_g(rows0, sg0)
    scale(rows0, (GPC - 1) * G)
    fire_s(GPC - 1, rows0, ss0)
    wait_s(rows0, ss0)
    wait_s(rows1, ss1)
    return 0

  lax.fori_loop(0, NCHK, chunk, 0)
  plsc.subcore_barrier()
  for i in range(ROWS_PT // ZROWS):
    pltpu.sync_copy(acc.at[pl.ds(s * ROWS_PT + i * ZROWS, ZROWS), :], zb)
    pltpu.sync_copy(zb, acc_out.at[c, pl.ds(s * ROWS_PT + i * ZROWS, ZROWS), :])


def _edge_call(d_feat, with_table):
  scratch = [
      pltpu.VMEM((CH,), jnp.int32),
      pltpu.VMEM((CH,), jnp.int32),
      pltpu.VMEM((CH,), jnp.int32),
      pltpu.VMEM((CH,), jnp.int32),
      pltpu.VMEM((GPC, G), jnp.int32),
      pltpu.VMEM((CH,), jnp.int32),
      pltpu.VMEM((CH,), jnp.float32),
      pltpu.VMEM((G, d_feat), jnp.float32),
      pltpu.VMEM((G, d_feat), jnp.float32),
      pltpu.VMEM((ZROWS, d_feat), jnp.float32),
      pltpu.VMEM_SHARED((N, d_feat), jnp.float32),
  ]
  if with_table:
    scratch.append(pltpu.VMEM((DEG_SLICE,), jnp.float32))
    scratch.append(pltpu.VMEM((G,), jnp.float32))
    scratch.append(pltpu.VMEM_SHARED((RNP,), jnp.float32))
  scratch += [
      pltpu.SemaphoreType.DMA,
      pltpu.SemaphoreType.DMA,
      pltpu.SemaphoreType.DMA,
      pltpu.SemaphoreType.DMA,
      pltpu.SemaphoreType.DMA,
  ]
  out_type = [jax.ShapeDtypeStruct((NC, N, d_feat), jnp.float32)]
  if with_table:
    out_type.append(jax.ShapeDtypeStruct((E,), jnp.float32))

  def body(*args):
    if with_table:
      (hall, src, dst, et, acc_out, norm_out,
       srcc, dstc, etc_, gidx, sidx, nidx, normc, rows0, rows1, zb, acc,
       dbuf, onesb, degacc, sg0, sg1, ss0, ss1, sn) = args
      recip = None
    else:
      (hall, src, dst, et, recip, acc_out,
       srcc, dstc, etc_, gidx, sidx, nidx, normc, rows0, rows1, zb, acc,
       sg0, sg1, ss0, ss1, sn) = args
      norm_out = dbuf = onesb = degacc = None
    _edge_body(d_feat, with_table, hall, src, dst, et, recip, acc_out,
               norm_out, srcc, dstc, etc_, gidx, sidx, nidx, normc, rows0,
               rows1, zb, acc, dbuf, onesb, degacc, sg0, sg1, ss0, ss1, sn)

  return pl.kernel(body, out_type=out_type, mesh=_mesh,
                   scratch_types=scratch, compiler_params=_sc_params)


_edge1_call = _edge_call(D_HID, True)
_edge2_call = _edge_call(D_OUT, False)


NB = 10
BN = N // NB


def _tc1_body(x_ref, w_ref, root_ref, b_ref, hall_ref, xr_ref):
  xb = x_ref[...]
  for r in range(R):
    hall_ref[r] = jnp.dot(xb, w_ref[r], preferred_element_type=jnp.float32)
  xr_ref[...] = (jnp.dot(xb, root_ref[...], preferred_element_type=jnp.float32)
                 + b_ref[...])


def _tc1(x, W1, root1, b1):
  return pl.pallas_call(
      _tc1_body,
      grid=(NB,),
      in_specs=[
          pl.BlockSpec((BN, D_IN), lambda i: (i, 0)),
          pl.BlockSpec((R, D_IN, D_HID), lambda i: (0, 0, 0)),
          pl.BlockSpec((D_IN, D_HID), lambda i: (0, 0)),
          pl.BlockSpec((1, D_HID), lambda i: (0, 0)),
      ],
      out_specs=[
          pl.BlockSpec((R, BN, D_HID), lambda i: (0, i, 0)),
          pl.BlockSpec((BN, D_HID), lambda i: (i, 0)),
      ],
      out_shape=[
          jax.ShapeDtypeStruct((R, N, D_HID), jnp.float32),
          jax.ShapeDtypeStruct((N, D_HID), jnp.float32),
      ],
  )(x, W1, root1, b1)


def _tc2_body(acc_ref, xr_ref, w_ref, root_ref, b_ref, hall_ref, xr2_ref):
  hb = jnp.maximum(acc_ref[0] + acc_ref[1] + xr_ref[...], 0.0)
  for r in range(R):
    hall_ref[r] = jnp.dot(hb, w_ref[r], preferred_element_type=jnp.float32)
  xr2_ref[...] = (jnp.dot(hb, root_ref[...], preferred_element_type=jnp.float32)
                  + b_ref[...])


def _tc2(acc1, xr1, W2, root2, b2):
  return pl.pallas_call(
      _tc2_body,
      grid=(NB,),
      in_specs=[
          pl.BlockSpec((2, BN, D_HID), lambda i: (0, i, 0)),
          pl.BlockSpec((BN, D_HID), lambda i: (i, 0)),
          pl.BlockSpec((R, D_HID, D_OUT), lambda i: (0, 0, 0)),
          pl.BlockSpec((D_HID, D_OUT), lambda i: (0, 0)),
          pl.BlockSpec((1, D_OUT), lambda i: (0, 0)),
      ],
      out_specs=[
          pl.BlockSpec((R, BN, D_OUT), lambda i: (0, i, 0)),
          pl.BlockSpec((BN, D_OUT), lambda i: (i, 0)),
      ],
      out_shape=[
          jax.ShapeDtypeStruct((R, N, D_OUT), jnp.float32),
          jax.ShapeDtypeStruct((N, D_OUT), jnp.float32),
      ],
  )(acc1, xr1, W2, root2, b2)


def _tc3_body(acc_ref, xr_ref, out_ref):
  out_ref[...] = acc_ref[0] + acc_ref[1] + xr_ref[...]


def _tc3(acc2, xr2):
  return pl.pallas_call(
      _tc3_body,
      grid=(NB,),
      in_specs=[
          pl.BlockSpec((2, BN, D_OUT), lambda i: (0, i, 0)),
          pl.BlockSpec((BN, D_OUT), lambda i: (i, 0)),
      ],
      out_specs=pl.BlockSpec((BN, D_OUT), lambda i: (i, 0)),
      out_shape=jax.ShapeDtypeStruct((N, D_OUT), jnp.float32),
  )(acc2, xr2)


@jax.jit
def kernel(x, edge_index, edge_type, W1, root1, b1, W2, root2, b2):
  src = edge_index[0]
  dst = edge_index[1]

  hall1, xr1 = _tc1(x, W1, root1, b1.reshape(1, D_HID))
  acc1, norm_e = _edge1_call(hall1.reshape(RN, D_HID), src, dst, edge_type)
  hall2, xr2 = _tc2(acc1, xr1, W2, root2, b2.reshape(1, D_OUT))
  (acc2,) = _edge2_call(hall2.reshape(RN, D_OUT), src, dst, edge_type, norm_e)
  return _tc3(acc2, xr2)
